# baseline (speedup 1.0000x reference)
import functools

import jax
import jax.numpy as jnp
from jax import lax
from jax.experimental import pallas as pl
from jax.experimental.pallas import tpu as pltpu
from jax.experimental.pallas import tpu_sc as plsc


def _assign_body(q_ref, k_ref, v_ref, w_ref, idx_ref, recip_ref, vvt_ref,
                 cnt_ref, *, n_total, mb_size):
    mb = pl.program_id(1)
    nm = pl.num_programs(1)
    q2 = q_ref[0]
    k2 = k_ref[0]
    logits = lax.dot_general(q2, k2, (((1,), (1,)), ((), ())),
                             preferred_element_type=jnp.float32)
    col_max = jnp.max(logits, axis=0)
    row_iota = lax.broadcasted_iota(jnp.int32, logits.shape, 0)
    is_max = logits == col_max[None, :]
    idx = jnp.min(jnp.where(is_max, row_iota, n_total), axis=0)
    valid = jnp.isfinite(col_max)
    idx = jnp.minimum(idx, n_total - 1)
    idx_ref[0, 0, pl.ds(mb * mb_size, mb_size)] = idx

    one_hot = ((row_iota == idx[None, :]) & valid[None, :]).astype(jnp.float32)
    cnts = jnp.sum(one_hot, axis=1, keepdims=True)

    vvt = lax.dot_general(w_ref[...], v_ref[0], (((1,), (1,)), ((), ())),
                          preferred_element_type=jnp.float32)
    vvt_ref[0] = vvt * valid[None, :].astype(jnp.float32)

    @pl.when(mb == 0)
    def _():
        cnt_ref[...] = cnts

    @pl.when(mb > 0)
    def _():
        cnt_ref[...] += cnts

    @pl.when(mb == nm - 1)
    def _():
        c = cnt_ref[...]
        recip_ref[0] = 1.0 / (jnp.maximum(c, 1.0) + 1e-6)


def _assign_project(q, k, v, W_v):
    B, N, D = q.shape
    M = k.shape[1]
    MB = min(512, M)
    body = functools.partial(_assign_body, n_total=N, mb_size=MB)
    return pl.pallas_call(
        body,
        grid=(B, M // MB),
        in_specs=[
            pl.BlockSpec((1, N, D), lambda b, m: (b, 0, 0)),
            pl.BlockSpec((1, MB, D), lambda b, m: (b, m, 0)),
            pl.BlockSpec((1, MB, D), lambda b, m: (b, m, 0)),
            pl.BlockSpec((D, D), lambda b, m: (0, 0)),
        ],
        out_specs=[
            pl.BlockSpec((1, 1, M), lambda b, m: (b, 0, 0)),
            pl.BlockSpec((1, N, 1), lambda b, m: (b, 0, 0)),
            pl.BlockSpec((1, D, MB), lambda b, m: (b, 0, m)),
        ],
        out_shape=[
            jax.ShapeDtypeStruct((B, 1, M), jnp.int32),
            jax.ShapeDtypeStruct((B, N, 1), jnp.float32),
            jax.ShapeDtypeStruct((B, D, M), jnp.float32),
        ],
        scratch_shapes=[pltpu.VMEM((N, 1), jnp.float32)],
    )(q, k, v, W_v)


_NTILE = 32
_MC = 512


def _make_sc_scatter(B, N, M, D):
    DT = D // _NTILE
    nchunk = M // _MC
    ngroup = _MC // 16
    mesh = plsc.VectorSubcoreMesh(core_axis_name="c", subcore_axis_name="s")

    n_slots = N
    UNROLL = 4

    @functools.partial(
        pl.kernel, mesh=mesh,
        compiler_params=pltpu.CompilerParams(needs_layout_passes=False),
        out_type=jax.ShapeDtypeStruct((B, _NTILE, 2, N * DT // 2),
                                      jnp.float32),
        scratch_types=[
            pltpu.VMEM((M,), jnp.int32),
            pltpu.VMEM((2, DT, _MC), jnp.float32),
            pltpu.VMEM((N * DT // 2,), jnp.float32),
            pltpu.VMEM((N * DT // 2,), jnp.float32),
            pltpu.SemaphoreType.DMA,
            pltpu.SemaphoreType.DMA,
            pltpu.SemaphoreType.DMA,
        ],
    )
    def sck(vvt_hbm, idx_hbm, zero_hbm, out_hbm, idx_v, buf, acc0, acc1,
            sem0, sem1, zsem):
        c = lax.axis_index("c")
        s = lax.axis_index("s")
        w = c * plsc.get_sparse_core_info().num_subcores + s
        sems = (sem0, sem1)
        accs = (acc0, acc1)
        half = DT // 2

        def gather(b, jc, p):
            return pltpu.async_copy(
                vvt_hbm.at[b, pl.ds(w * DT, DT), pl.ds(jc * _MC, _MC)],
                buf.at[p], sems[p])

        for b in range(B):
            zcopy0 = pltpu.async_copy(zero_hbm, acc0, zsem)
            zcopy1 = pltpu.async_copy(zero_hbm, acc1, zsem)
            pltpu.sync_copy(idx_hbm.at[b], idx_v)
            g0 = gather(b, 0, 0)
            zcopy0.wait()
            zcopy1.wait()
            for jc in range(nchunk):
                gnext = gather(b, jc + 1, (jc + 1) % 2) if jc + 1 < nchunk \
                    else None
                (g0 if jc == 0 else gprev).wait()
                p = jc % 2

                def gbody(g4, carry, jc=jc, p=p):
                    for u in range(UNROLL):
                        g = g4 * UNROLL + u
                        rows = idx_v[pl.ds(jc * _MC + g * 16, 16)]
                        for cc in range(half):
                            plsc.addupdate_scatter(
                                accs[0], [rows + cc * n_slots],
                                buf[p, cc, pl.ds(g * 16, 16)])
                            plsc.addupdate_scatter(
                                accs[1], [rows + cc * n_slots],
                                buf[p, half + cc, pl.ds(g * 16, 16)])
                    return carry
                lax.fori_loop(0, ngroup // UNROLL, gbody, 0)
                gprev = gnext
            pltpu.sync_copy(acc0, out_hbm.at[b, w, 0])
            pltpu.sync_copy(acc1, out_hbm.at[b, w, 1])

    return sck


def _norm_body(raw_ref, recip_ref, out_ref, *, slabs, DT):
    ii = lax.broadcasted_iota(jnp.int32, (DT, DT), 0)
    jj = lax.broadcasted_iota(jnp.int32, (DT, DT), 1)
    eye = (ii == jj).astype(jnp.float32)
    raw4 = raw_ref[...]
    parts = [
        lax.dot_general(raw4[i], eye, (((0,), (0,)), ((), ())),
                        preferred_element_type=jnp.float32)
        for i in range(slabs)
    ]
    merged = jnp.concatenate(parts, axis=1)
    out_ref[0] = merged * recip_ref[0]


def _normalize(raw, recip, D):
    bnt, DT, N = raw.shape
    slabs = 128 // DT
    B = recip.shape[0]
    ntile = bnt // B
    nw = ntile // slabs
    body = functools.partial(_norm_body, slabs=slabs, DT=DT)
    return pl.pallas_call(
        body,
        grid=(B, nw),
        in_specs=[
            pl.BlockSpec((slabs, DT, N), lambda b, w: (b * nw + w, 0, 0)),
            pl.BlockSpec((1, N, 1), lambda b, w: (b, 0, 0)),
        ],
        out_specs=pl.BlockSpec((1, N, slabs * DT), lambda b, w: (b, 0, w)),
        out_shape=jax.ShapeDtypeStruct((B, N, D), jnp.float32),
    )(raw, recip)


def kernel(q, k, v, W_v):
    B, N, D = q.shape
    M = k.shape[1]
    idx, recip, vvt = _assign_project(q, k, v, W_v)

    DT = D // _NTILE
    sck = _make_sc_scatter(B, N, M, D)
    zeros = jnp.zeros((N * DT // 2,), jnp.float32)
    raw = sck(vvt, idx.reshape(B, M), zeros)
    return _normalize(raw.reshape(B * _NTILE, DT, N), recip, D)


# speedup vs baseline: 1.2077x; 1.2077x over previous
import functools

import jax
import jax.numpy as jnp
from jax import lax
from jax.experimental import pallas as pl
from jax.experimental.pallas import tpu as pltpu
from jax.experimental.pallas import tpu_sc as plsc


def _assign_body(q_ref, k_ref, v_ref, w_ref, idx_ref, recip_ref, vvt_ref,
                 cnt_ref, *, n_total, mb_size):
    mb = pl.program_id(1)
    nm = pl.num_programs(1)
    q2 = q_ref[0]
    k2 = k_ref[0]
    logits = lax.dot_general(q2, k2, (((1,), (1,)), ((), ())),
                             preferred_element_type=jnp.float32)
    col_max = jnp.max(logits, axis=0)
    row_iota = lax.broadcasted_iota(jnp.int32, logits.shape, 0)
    is_max = logits == col_max[None, :]
    idx = jnp.min(jnp.where(is_max, row_iota, n_total), axis=0)
    valid = jnp.isfinite(col_max)
    idx = jnp.minimum(idx, n_total - 1)
    idx_ref[0, 0, pl.ds(mb * mb_size, mb_size)] = idx

    one_hot = ((row_iota == idx[None, :]) & valid[None, :]).astype(jnp.float32)
    cnts = jnp.sum(one_hot, axis=1, keepdims=True)

    vvt = lax.dot_general(w_ref[...], v_ref[0], (((1,), (1,)), ((), ())),
                          preferred_element_type=jnp.float32)
    vvt_ref[0] = vvt * valid[None, :].astype(jnp.float32)

    @pl.when(mb == 0)
    def _():
        cnt_ref[...] = cnts

    @pl.when(mb > 0)
    def _():
        cnt_ref[...] += cnts

    @pl.when(mb == nm - 1)
    def _():
        c = cnt_ref[...]
        recip_ref[0] = 1.0 / (jnp.maximum(c, 1.0) + 1e-6)


def _assign_project(q, k, v, W_v):
    B, N, D = q.shape
    M = k.shape[1]
    MB = min(512, M)
    body = functools.partial(_assign_body, n_total=N, mb_size=MB)
    return pl.pallas_call(
        body,
        grid=(B, M // MB),
        in_specs=[
            pl.BlockSpec((1, N, D), lambda b, m: (b, 0, 0)),
            pl.BlockSpec((1, MB, D), lambda b, m: (b, m, 0)),
            pl.BlockSpec((1, MB, D), lambda b, m: (b, m, 0)),
            pl.BlockSpec((D, D), lambda b, m: (0, 0)),
        ],
        out_specs=[
            pl.BlockSpec((1, 1, M), lambda b, m: (b, 0, 0)),
            pl.BlockSpec((1, N, 1), lambda b, m: (b, 0, 0)),
            pl.BlockSpec((1, D, MB), lambda b, m: (b, 0, m)),
        ],
        out_shape=[
            jax.ShapeDtypeStruct((B, 1, M), jnp.int32),
            jax.ShapeDtypeStruct((B, N, 1), jnp.float32),
            jax.ShapeDtypeStruct((B, D, M), jnp.float32),
        ],
        scratch_shapes=[pltpu.VMEM((N, 1), jnp.float32)],
    )(q, k, v, W_v)


_NTILE = 32
_MC = 512


def _make_sc_scatter(B, N, M, D):
    DT = D // _NTILE
    nchunk = M // _MC
    ngroup = _MC // 16
    mesh = plsc.VectorSubcoreMesh(core_axis_name="c", subcore_axis_name="s")

    n_slots = N
    UNROLL = 4

    @functools.partial(
        pl.kernel, mesh=mesh,
        compiler_params=pltpu.CompilerParams(needs_layout_passes=False),
        out_type=jax.ShapeDtypeStruct((B, _NTILE, N * DT), jnp.float32),
        scratch_types=[
            pltpu.VMEM((M,), jnp.int32),
            pltpu.VMEM((2, DT, _MC), jnp.float32),
            pltpu.VMEM((N * DT,), jnp.float32),
            pltpu.SemaphoreType.DMA,
            pltpu.SemaphoreType.DMA,
            pltpu.SemaphoreType.DMA,
        ],
    )
    def sck(vvt_hbm, idx_hbm, zero_hbm, out_hbm, idx_v, buf, acc,
            sem0, sem1, zsem):
        c = lax.axis_index("c")
        s = lax.axis_index("s")
        w = c * plsc.get_sparse_core_info().num_subcores + s
        sems = (sem0, sem1)

        def gather(b, jc, p):
            return pltpu.async_copy(
                vvt_hbm.at[b, pl.ds(w * DT, DT), pl.ds(jc * _MC, _MC)],
                buf.at[p], sems[p])

        for b in range(B):
            zcopy = pltpu.async_copy(zero_hbm, acc, zsem)
            pltpu.sync_copy(idx_hbm.at[b], idx_v)
            g0 = gather(b, 0, 0)
            zcopy.wait()
            for jc in range(nchunk):
                gnext = gather(b, jc + 1, (jc + 1) % 2) if jc + 1 < nchunk \
                    else None
                (g0 if jc == 0 else gprev).wait()
                p = jc % 2

                def gbody(g4, carry, jc=jc, p=p):
                    for u in range(UNROLL):
                        g = g4 * UNROLL + u
                        rows = idx_v[pl.ds(jc * _MC + g * 16, 16)]
                        vals = [buf[p, cc, pl.ds(g * 16, 16)]
                                for cc in range(DT)]
                        addrs = [rows + cc * n_slots for cc in range(DT)]
                        for cc in range(DT):
                            plsc.addupdate_scatter(
                                acc, [addrs[cc]], vals[cc])
                    return carry
                lax.fori_loop(0, ngroup // UNROLL, gbody, 0)
                gprev = gnext
            pltpu.sync_copy(acc, out_hbm.at[b, w])

    return sck


def _norm_body(raw_ref, recip_ref, out_ref, *, slabs, DT):
    ii = lax.broadcasted_iota(jnp.int32, (DT, DT), 0)
    jj = lax.broadcasted_iota(jnp.int32, (DT, DT), 1)
    eye = (ii == jj).astype(jnp.float32)
    raw4 = raw_ref[...]
    parts = [
        lax.dot_general(raw4[i], eye, (((0,), (0,)), ((), ())),
                        preferred_element_type=jnp.float32)
        for i in range(slabs)
    ]
    merged = jnp.concatenate(parts, axis=1)
    out_ref[0] = merged * recip_ref[0]


def _normalize(raw, recip, D):
    bnt, DT, N = raw.shape
    slabs = 128 // DT
    B = recip.shape[0]
    ntile = bnt // B
    nw = ntile // slabs
    body = functools.partial(_norm_body, slabs=slabs, DT=DT)
    return pl.pallas_call(
        body,
        grid=(B, nw),
        in_specs=[
            pl.BlockSpec((slabs, DT, N), lambda b, w: (b * nw + w, 0, 0)),
            pl.BlockSpec((1, N, 1), lambda b, w: (b, 0, 0)),
        ],
        out_specs=pl.BlockSpec((1, N, slabs * DT), lambda b, w: (b, 0, w)),
        out_shape=jax.ShapeDtypeStruct((B, N, D), jnp.float32),
    )(raw, recip)


def kernel(q, k, v, W_v):
    B, N, D = q.shape
    M = k.shape[1]
    idx, recip, vvt = _assign_project(q, k, v, W_v)

    DT = D // _NTILE
    sck = _make_sc_scatter(B, N, M, D)
    zeros = jnp.zeros((N * DT,), jnp.float32)
    raw = sck(vvt, idx.reshape(B, M), zeros)
    return _normalize(raw.reshape(B * _NTILE, DT, N), recip, D)


# speedup vs baseline: 1.3086x; 1.0835x over previous
import functools

import jax
import jax.numpy as jnp
from jax import lax
from jax.experimental import pallas as pl
from jax.experimental.pallas import tpu as pltpu
from jax.experimental.pallas import tpu_sc as plsc


def _assign_body(q_ref, k_ref, v_ref, w_ref, idx_ref, recip_ref, vvt_ref,
                 cnt_ref, *, n_total, mb_size):
    mb = pl.program_id(1)
    nm = pl.num_programs(1)
    q2 = q_ref[0]
    k2 = k_ref[0]
    logits = lax.dot_general(q2, k2, (((1,), (1,)), ((), ())),
                             preferred_element_type=jnp.float32)
    col_max = jnp.max(logits, axis=0)
    row_iota = lax.broadcasted_iota(jnp.int32, logits.shape, 0)
    is_max = logits == col_max[None, :]
    idx = jnp.min(jnp.where(is_max, row_iota, n_total), axis=0)
    valid = jnp.isfinite(col_max)
    idx = jnp.minimum(idx, n_total - 1)
    idx_ref[0, 0, pl.ds(mb * mb_size, mb_size)] = idx

    one_hot = ((row_iota == idx[None, :]) & valid[None, :]).astype(jnp.float32)
    cnts = jnp.sum(one_hot, axis=1, keepdims=True)

    vvt = lax.dot_general(w_ref[...], v_ref[0], (((1,), (1,)), ((), ())),
                          preferred_element_type=jnp.float32)
    vvt_ref[0] = vvt * valid[None, :].astype(jnp.float32)

    @pl.when(mb == 0)
    def _():
        cnt_ref[...] = cnts

    @pl.when(mb > 0)
    def _():
        cnt_ref[...] += cnts

    @pl.when(mb == nm - 1)
    def _():
        c = cnt_ref[...]
        recip_ref[0] = 1.0 / (jnp.maximum(c, 1.0) + 1e-6)


def _assign_project(q, k, v, W_v):
    B, N, D = q.shape
    M = k.shape[1]
    MB = min(512, M)
    body = functools.partial(_assign_body, n_total=N, mb_size=MB)
    return pl.pallas_call(
        body,
        grid=(B, M // MB),
        in_specs=[
            pl.BlockSpec((1, N, D), lambda b, m: (b, 0, 0)),
            pl.BlockSpec((1, MB, D), lambda b, m: (b, m, 0)),
            pl.BlockSpec((1, MB, D), lambda b, m: (b, m, 0)),
            pl.BlockSpec((D, D), lambda b, m: (0, 0)),
        ],
        out_specs=[
            pl.BlockSpec((1, 1, M), lambda b, m: (b, 0, 0)),
            pl.BlockSpec((1, N, 1), lambda b, m: (b, 0, 0)),
            pl.BlockSpec((1, D, MB), lambda b, m: (b, 0, m)),
        ],
        out_shape=[
            jax.ShapeDtypeStruct((B, 1, M), jnp.int32),
            jax.ShapeDtypeStruct((B, N, 1), jnp.float32),
            jax.ShapeDtypeStruct((B, D, M), jnp.float32),
        ],
        scratch_shapes=[pltpu.VMEM((N, 1), jnp.float32)],
    )(q, k, v, W_v)


_NTILE = 32
_MC = 512


def _make_sc_scatter(B, N, M, D):
    DT = D // _NTILE
    nchunk = M // _MC
    ngroup = _MC // 16
    mesh = plsc.VectorSubcoreMesh(core_axis_name="c", subcore_axis_name="s")

    n_slots = N
    UNROLL = 4

    @functools.partial(
        pl.kernel, mesh=mesh,
        compiler_params=pltpu.CompilerParams(needs_layout_passes=False),
        out_type=jax.ShapeDtypeStruct((B * _NTILE, DT, N), jnp.float32),
        scratch_types=[
            pltpu.VMEM((M,), jnp.int32),
            pltpu.VMEM((2, DT, _MC), jnp.float32),
            pltpu.VMEM((DT, N), jnp.float32),
            pltpu.SemaphoreType.DMA,
            pltpu.SemaphoreType.DMA,
            pltpu.SemaphoreType.DMA,
        ],
    )
    def sck(vvt_hbm, idx_hbm, zero_hbm, out_hbm, idx_v, buf, acc,
            sem0, sem1, zsem):
        c = lax.axis_index("c")
        s = lax.axis_index("s")
        w = c * plsc.get_sparse_core_info().num_subcores + s
        sems = (sem0, sem1)

        def gather(b, jc, p):
            return pltpu.async_copy(
                vvt_hbm.at[b, pl.ds(w * DT, DT), pl.ds(jc * _MC, _MC)],
                buf.at[p], sems[p])

        for b in range(B):
            zcopy = pltpu.async_copy(zero_hbm, acc, zsem)
            pltpu.sync_copy(idx_hbm.at[b, 0], idx_v)
            g0 = gather(b, 0, 0)
            zcopy.wait()
            for jc in range(nchunk):
                gnext = gather(b, jc + 1, (jc + 1) % 2) if jc + 1 < nchunk \
                    else None
                (g0 if jc == 0 else gprev).wait()
                p = jc % 2

                def gbody(g4, carry, jc=jc, p=p):
                    for u in range(UNROLL):
                        g = g4 * UNROLL + u
                        rows = idx_v[pl.ds(jc * _MC + g * 16, 16)]
                        vals = [buf[p, cc, pl.ds(g * 16, 16)]
                                for cc in range(DT)]
                        cols = [jnp.full((16,), cc, jnp.int32)
                                for cc in range(DT)]
                        for cc in range(DT):
                            plsc.addupdate_scatter(
                                acc, [cols[cc], rows], vals[cc])
                    return carry
                lax.fori_loop(0, ngroup // UNROLL, gbody, 0)
                gprev = gnext
            pltpu.sync_copy(acc, out_hbm.at[b * _NTILE + w])

    return sck


def _norm_body(raw_ref, recip_ref, out_ref, *, slabs, DT):
    ii = lax.broadcasted_iota(jnp.int32, (DT, DT), 0)
    jj = lax.broadcasted_iota(jnp.int32, (DT, DT), 1)
    eye = (ii == jj).astype(jnp.float32)
    raw4 = raw_ref[...]
    parts = [
        lax.dot_general(raw4[i], eye, (((0,), (0,)), ((), ())),
                        preferred_element_type=jnp.float32)
        for i in range(slabs)
    ]
    merged = jnp.concatenate(parts, axis=1)
    out_ref[0] = merged * recip_ref[0]


def _normalize(raw, recip, D):
    bnt, DT, N = raw.shape
    slabs = 128 // DT
    B = recip.shape[0]
    ntile = bnt // B
    nw = ntile // slabs
    body = functools.partial(_norm_body, slabs=slabs, DT=DT)
    return pl.pallas_call(
        body,
        grid=(B, nw),
        in_specs=[
            pl.BlockSpec((slabs, DT, N), lambda b, w: (b * nw + w, 0, 0)),
            pl.BlockSpec((1, N, 1), lambda b, w: (b, 0, 0)),
        ],
        out_specs=pl.BlockSpec((1, N, slabs * DT), lambda b, w: (b, 0, w)),
        out_shape=jax.ShapeDtypeStruct((B, N, D), jnp.float32),
    )(raw, recip)


def kernel(q, k, v, W_v):
    B, N, D = q.shape
    M = k.shape[1]
    idx, recip, vvt = _assign_project(q, k, v, W_v)

    DT = D // _NTILE
    sck = _make_sc_scatter(B, N, M, D)
    zeros = jnp.zeros((DT, N), jnp.float32)
    raw = sck(vvt, idx, zeros)
    return _normalize(raw, recip, D)


# speedup vs baseline: 1.3639x; 1.0423x over previous
import functools

import jax
import jax.numpy as jnp
from jax import lax
from jax.experimental import pallas as pl
from jax.experimental.pallas import tpu as pltpu
from jax.experimental.pallas import tpu_sc as plsc


def _assign_body(q_ref, k_ref, v_ref, w_ref, idx_ref, recip_ref, vvt_ref,
                 cnt_ref, *, n_total, mb_size):
    mb = pl.program_id(1)
    nm = pl.num_programs(1)
    q2 = q_ref[0]
    k2 = k_ref[0]
    logits = lax.dot_general(q2, k2, (((1,), (1,)), ((), ())),
                             preferred_element_type=jnp.float32)
    col_max = jnp.max(logits, axis=0)
    row_iota = lax.broadcasted_iota(jnp.int32, logits.shape, 0)
    is_max = logits == col_max[None, :]
    idx = jnp.min(jnp.where(is_max, row_iota, n_total), axis=0)
    valid = jnp.isfinite(col_max)
    idx = jnp.minimum(idx, n_total - 1)
    idx_ref[0, 0, pl.ds(mb * mb_size, mb_size)] = idx

    one_hot = ((row_iota == idx[None, :]) & valid[None, :]).astype(jnp.float32)
    cnts = jnp.sum(one_hot, axis=1, keepdims=True)

    vvt = lax.dot_general(w_ref[...], v_ref[0], (((1,), (1,)), ((), ())),
                          preferred_element_type=jnp.float32)
    vvt_ref[0] = vvt * valid[None, :].astype(jnp.float32)

    @pl.when(mb == 0)
    def _():
        cnt_ref[...] = cnts

    @pl.when(mb > 0)
    def _():
        cnt_ref[...] += cnts

    @pl.when(mb == nm - 1)
    def _():
        c = cnt_ref[...]
        recip_ref[0] = 1.0 / (jnp.maximum(c, 1.0) + 1e-6)


def _assign_project(q, k, v, W_v):
    B, N, D = q.shape
    M = k.shape[1]
    MB = min(1024, M)
    body = functools.partial(_assign_body, n_total=N, mb_size=MB)
    return pl.pallas_call(
        body,
        grid=(B, M // MB),
        in_specs=[
            pl.BlockSpec((1, N, D), lambda b, m: (b, 0, 0)),
            pl.BlockSpec((1, MB, D), lambda b, m: (b, m, 0)),
            pl.BlockSpec((1, MB, D), lambda b, m: (b, m, 0)),
            pl.BlockSpec((D, D), lambda b, m: (0, 0)),
        ],
        out_specs=[
            pl.BlockSpec((1, 1, M), lambda b, m: (b, 0, 0)),
            pl.BlockSpec((1, N, 1), lambda b, m: (b, 0, 0)),
            pl.BlockSpec((1, D, MB), lambda b, m: (b, 0, m)),
        ],
        out_shape=[
            jax.ShapeDtypeStruct((B, 1, M), jnp.int32),
            jax.ShapeDtypeStruct((B, N, 1), jnp.float32),
            jax.ShapeDtypeStruct((B, D, M), jnp.float32),
        ],
        scratch_shapes=[pltpu.VMEM((N, 1), jnp.float32)],
    )(q, k, v, W_v)


_NTILE = 32
_MC = 512


def _make_sc_scatter(B, N, M, D):
    DT = D // _NTILE
    nchunk = M // _MC
    ngroup = _MC // 16
    mesh = plsc.VectorSubcoreMesh(core_axis_name="c", subcore_axis_name="s")

    n_slots = N
    UNROLL = 4

    @functools.partial(
        pl.kernel, mesh=mesh,
        compiler_params=pltpu.CompilerParams(needs_layout_passes=False),
        out_type=jax.ShapeDtypeStruct((B * _NTILE, DT, N), jnp.float32),
        scratch_types=[
            pltpu.VMEM((M,), jnp.int32),
            pltpu.VMEM((2, DT, _MC), jnp.float32),
            pltpu.VMEM((DT, N), jnp.float32),
            pltpu.SemaphoreType.DMA,
            pltpu.SemaphoreType.DMA,
            pltpu.SemaphoreType.DMA,
        ],
    )
    def sck(vvt_hbm, idx_hbm, zero_hbm, out_hbm, idx_v, buf, acc,
            sem0, sem1, zsem):
        c = lax.axis_index("c")
        s = lax.axis_index("s")
        w = c * plsc.get_sparse_core_info().num_subcores + s
        sems = (sem0, sem1)

        def gather(b, jc, p):
            return pltpu.async_copy(
                vvt_hbm.at[b, pl.ds(w * DT, DT), pl.ds(jc * _MC, _MC)],
                buf.at[p], sems[p])

        for b in range(B):
            zcopy = pltpu.async_copy(zero_hbm, acc, zsem)
            pltpu.sync_copy(idx_hbm.at[b, 0], idx_v)
            g0 = gather(b, 0, 0)
            zcopy.wait()
            for jc in range(nchunk):
                gnext = gather(b, jc + 1, (jc + 1) % 2) if jc + 1 < nchunk \
                    else None
                (g0 if jc == 0 else gprev).wait()
                p = jc % 2

                def gbody(g4, carry, jc=jc, p=p):
                    for u in range(UNROLL):
                        g = g4 * UNROLL + u
                        rows = idx_v[pl.ds(jc * _MC + g * 16, 16)]
                        vals = [buf[p, cc, pl.ds(g * 16, 16)]
                                for cc in range(DT)]
                        cols = [jnp.full((16,), cc, jnp.int32)
                                for cc in range(DT)]
                        for cc in range(DT):
                            plsc.addupdate_scatter(
                                acc, [cols[cc], rows], vals[cc])
                    return carry
                lax.fori_loop(0, ngroup // UNROLL, gbody, 0)
                gprev = gnext
            pltpu.sync_copy(acc, out_hbm.at[b * _NTILE + w])

    return sck


def _norm_body(raw_ref, recip_ref, out_ref, *, slabs, DT):
    ii = lax.broadcasted_iota(jnp.int32, (DT, DT), 0)
    jj = lax.broadcasted_iota(jnp.int32, (DT, DT), 1)
    eye = (ii == jj).astype(jnp.float32)
    raw4 = raw_ref[...]
    parts = [
        lax.dot_general(raw4[i], eye, (((0,), (0,)), ((), ())),
                        preferred_element_type=jnp.float32)
        for i in range(slabs)
    ]
    merged = jnp.concatenate(parts, axis=1)
    out_ref[0] = merged * recip_ref[0]


def _normalize(raw, recip, D):
    bnt, DT, N = raw.shape
    slabs = 128 // DT
    B = recip.shape[0]
    ntile = bnt // B
    nw = ntile // slabs
    body = functools.partial(_norm_body, slabs=slabs, DT=DT)
    return pl.pallas_call(
        body,
        grid=(B, nw),
        in_specs=[
            pl.BlockSpec((slabs, DT, N), lambda b, w: (b * nw + w, 0, 0)),
            pl.BlockSpec((1, N, 1), lambda b, w: (b, 0, 0)),
        ],
        out_specs=pl.BlockSpec((1, N, slabs * DT), lambda b, w: (b, 0, w)),
        out_shape=jax.ShapeDtypeStruct((B, N, D), jnp.float32),
    )(raw, recip)


def kernel(q, k, v, W_v):
    B, N, D = q.shape
    M = k.shape[1]
    idx, recip, vvt = _assign_project(q, k, v, W_v)

    DT = D // _NTILE
    sck = _make_sc_scatter(B, N, M, D)
    zeros = jnp.zeros((DT, N), jnp.float32)
    raw = sck(vvt, idx, zeros)
    return _normalize(raw, recip, D)
